# baseline (speedup 1.0000x reference)
import functools

import jax
import jax.numpy as jnp
from jax import lax
from jax.experimental import pallas as pl
from jax.experimental.pallas import tpu as pltpu
from jax.experimental.pallas import tpu_sc as plsc

N, C, H, W = 2, 256, 100, 100
R = 512
PH = PW = 7
SR = 2
SCALE = 0.25

NCORES = 2
NSUB = 16
NW = NCORES * NSUB
RPW = R // NW
NSAMP = PH * SR
NBIN = PH * PW
NCHUNK = 13
CV = C // 16


TOT = RPW * NCHUNK


def _roi_align_body(flat, roisp, out, roi_v, bref, ylw, yhw, xli,
                    lyf, hyf, lxf, hxf, wbuf, rows, acc, sems, sem_out):
    cid = lax.axis_index("c")
    sid = lax.axis_index("s")
    wid = sid * NCORES + cid
    roi_base = wid * RPW

    pltpu.sync_copy(roisp.at[wid], roi_v)

    io = lax.iota(jnp.int32, 16)
    iof = io.astype(jnp.float32)
    coeff = iof * 0.5 + 0.25
    lane_ok = io < NSAMP

    def splat(v):
        return jnp.full((16,), v, jnp.int32)

    def axis_quantities(start, binsz, extent):
        ss = start + coeff * binsz
        valid = (ss >= -1.0) & (ss <= float(extent)) & lane_ok
        s0 = jnp.maximum(ss, 0.0)
        lo = jnp.minimum(s0.astype(jnp.int32), extent - 1)
        hi = jnp.minimum(lo + 1, extent - 1)
        frac = jnp.minimum(s0, float(extent - 1)) - lo.astype(jnp.float32)
        lofrac = jnp.where(valid, 1.0 - frac, 0.0)
        hifrac = jnp.where(valid, frac, 0.0)
        return lo, hi, lofrac, hifrac

    def setup_roi(r, _):
        def field(i):
            return plsc.load_gather(roi_v, [splat(i), splat(r)])

        b_v = field(0).astype(jnp.int32) * (H * W)
        sx_s = field(1) * SCALE - 0.5
        sy_s = field(2) * SCALE - 0.5
        ex_s = field(3) * SCALE - 0.5
        ey_s = field(4) * SCALE - 0.5
        bw_s = (ex_s - sx_s) / PW
        bh_s = (ey_s - sy_s) / PH
        yl_v, yh_v, hy_v, ly_v = axis_quantities(sy_s, bh_s, H)
        xl_v, _xh_v, hx_v, lx_v = axis_quantities(sx_s, bw_s, W)
        bref[r] = b_v
        ylw[r] = yl_v * W
        yhw[r] = yh_v * W
        xli[r] = xl_v
        lyf[r] = ly_v
        hyf[r] = hy_v
        lxf[r] = lx_v * 0.25
        hxf[r] = hx_v * 0.25
        return 0

    lax.fori_loop(0, RPW, setup_roi, 0)

    def rr_q(k):
        rr = (k * 5042) >> 16
        return rr, k - rr * NCHUNK

    def issue(k, slot):
        rr, q = rr_q(k)
        rsp = splat(rr)
        b_v = bref[rr]
        s = q * 16 + io
        bin_ = s >> 2
        rem = s & 3
        py = (bin_ * 9363) >> 16
        px = bin_ - py * 7
        ky = py * 2 + (rem >> 1)
        kx = px * 2 + (rem & 1)
        ylw_s = plsc.load_gather(ylw, [rsp, ky])
        yhw_s = plsc.load_gather(yhw, [rsp, ky])
        hy_s = plsc.load_gather(hyf, [rsp, ky])
        ly_s = plsc.load_gather(lyf, [rsp, ky])
        xl_s = plsc.load_gather(xli, [rsp, kx])
        hx_s = plsc.load_gather(hxf, [rsp, kx])
        lx_s = plsc.load_gather(lxf, [rsp, kx])
        wbuf[slot, 0] = hy_s * hx_s
        wbuf[slot, 1] = hy_s * lx_s
        wbuf[slot, 2] = ly_s * hx_s
        wbuf[slot, 3] = ly_s * lx_s
        ilo = b_v + ylw_s + xl_s
        ihi = b_v + yhw_s + xl_s
        sem = sems.at[slot]
        pltpu.make_async_copy(flat.at[ilo], rows.at[slot, pl.ds(0, 16)], sem).start()
        pltpu.make_async_copy(flat.at[ihi], rows.at[slot, pl.ds(16, 16)], sem).start()

    sh16 = jnp.full((16,), 16, jnp.int32)
    himask = jnp.full((16,), -65536, jnp.int32)

    def step(k, _):
        slot = k & 3
        rr, q = rr_q(k)
        aslot = rr & 1

        @pl.when(k < TOT - 2)
        def _():
            issue(k + 2, (k + 2) & 3)

        @pl.when((q == 0) & (rr >= 2))
        def _():
            pltpu.make_async_copy(acc.at[aslot], out.at[roi_base + rr],
                                  sem_out).wait()

        pltpu.make_async_copy(flat.at[pl.ds(0, 32)], rows.at[slot],
                              sems.at[slot]).wait()

        for t in range(4):
            acce = [jnp.zeros((16,), jnp.float32) for _ in range(8)]
            acco = [jnp.zeros((16,), jnp.float32) for _ in range(8)]
            for i in range(4):
                w1 = plsc.load_gather(wbuf, [splat(slot), splat(0), splat(t * 4 + i)])
                w2 = plsc.load_gather(wbuf, [splat(slot), splat(1), splat(t * 4 + i)])
                w3 = plsc.load_gather(wbuf, [splat(slot), splat(2), splat(t * 4 + i)])
                w4 = plsc.load_gather(wbuf, [splat(slot), splat(3), splat(t * 4 + i)])
                rbl = t * 4 + i
                rbh = 16 + t * 4 + i
                for g in range(8):
                    u1 = rows[slot, rbl, pl.ds(g * 16, 16)]
                    u2 = rows[slot, rbl, pl.ds(128 + g * 16, 16)]
                    u3 = rows[slot, rbh, pl.ds(g * 16, 16)]
                    u4 = rows[slot, rbh, pl.ds(128 + g * 16, 16)]
                    acce[g] = (acce[g]
                               + w1 * plsc.bitcast(u1 << sh16, jnp.float32)
                               + w2 * plsc.bitcast(u2 << sh16, jnp.float32)
                               + w3 * plsc.bitcast(u3 << sh16, jnp.float32)
                               + w4 * plsc.bitcast(u4 << sh16, jnp.float32))
                    acco[g] = (acco[g]
                               + w1 * plsc.bitcast(u1 & himask, jnp.float32)
                               + w2 * plsc.bitcast(u2 & himask, jnp.float32)
                               + w3 * plsc.bitcast(u3 & himask, jnp.float32)
                               + w4 * plsc.bitcast(u4 & himask, jnp.float32))
            binrow = q * 4 + t
            for g in range(8):
                acc[aslot, binrow, pl.ds(g * 16, 16)] = acce[g]
                acc[aslot, binrow, pl.ds(128 + g * 16, 16)] = acco[g]

        @pl.when(q == NCHUNK - 1)
        def _():
            pltpu.make_async_copy(acc.at[aslot], out.at[roi_base + rr],
                                  sem_out).start()

        return 0

    issue(0, 0)
    issue(1, 1)
    lax.fori_loop(0, TOT, step, 0)
    pltpu.make_async_copy(acc.at[0], out.at[roi_base], sem_out).wait()
    pltpu.make_async_copy(acc.at[1], out.at[roi_base], sem_out).wait()


def _prep_body(in_ref, out_ref):
    t = in_ref[0].reshape(C, H * W).T.astype(jnp.bfloat16)
    lo = lax.bitcast_convert_type(t[:, :C // 2], jnp.uint16).astype(jnp.int32)
    hi = lax.bitcast_convert_type(t[:, C // 2:], jnp.uint16).astype(jnp.int32)
    packed = lo | (hi << 16)
    out_ref[:, :C // 2] = packed
    out_ref[:, C // 2:] = jnp.concatenate([packed[1:], packed[:1]], axis=0)


@jax.jit
def _prep_tc(x):
    return pl.pallas_call(
        _prep_body,
        grid=(N,),
        in_specs=[pl.BlockSpec((1, C, H, W), lambda b: (b, 0, 0, 0))],
        out_specs=pl.BlockSpec((H * W, C), lambda b: (b, 0)),
        out_shape=jax.ShapeDtypeStruct((N * H * W, C), jnp.int32),
    )(x)


RB = 8


def _fixup_body(in_ref, out_ref):
    for i in range(RB):
        out_ref[i] = in_ref[i, :NBIN, :].T


@jax.jit
def _fixup_tc(x):
    return pl.pallas_call(
        _fixup_body,
        grid=(R // RB,),
        in_specs=[pl.BlockSpec((RB, 52, C), lambda b: (b, 0, 0))],
        out_specs=pl.BlockSpec((RB, C, NBIN), lambda b: (b, 0, 0)),
        out_shape=jax.ShapeDtypeStruct((R, C, NBIN), jnp.float32),
    )(x)


@jax.jit
def _roi_align_sc(flat, roisp):
    mesh = plsc.VectorSubcoreMesh(core_axis_name="c", subcore_axis_name="s",
                                  num_cores=NCORES, num_subcores=NSUB)
    run = pl.kernel(
        _roi_align_body,
        out_type=jax.ShapeDtypeStruct((R, 52, C), jnp.float32),
        mesh=mesh,
        scratch_types=[
            pltpu.VMEM((5, RPW), jnp.float32),
            pltpu.VMEM((RPW, 16), jnp.int32),
            pltpu.VMEM((RPW, 16), jnp.int32),
            pltpu.VMEM((RPW, 16), jnp.int32),
            pltpu.VMEM((RPW, 16), jnp.int32),
            pltpu.VMEM((RPW, 16), jnp.float32),
            pltpu.VMEM((RPW, 16), jnp.float32),
            pltpu.VMEM((RPW, 16), jnp.float32),
            pltpu.VMEM((RPW, 16), jnp.float32),
            pltpu.VMEM((4, 4, 16), jnp.float32),
            pltpu.VMEM((4, 32, C), jnp.int32),
            pltpu.VMEM((2, 52, C), jnp.float32),
            pltpu.SemaphoreType.DMA((4,)),
            pltpu.SemaphoreType.DMA,
        ],
        compiler_params=pltpu.CompilerParams(needs_layout_passes=False),
    )
    return run(flat, roisp)


def kernel(input, rois):
    flat = _prep_tc(input)
    roisp = jnp.transpose(rois, (1, 0)).reshape(5, NW, RPW).transpose(1, 0, 2)
    out = _roi_align_sc(flat, roisp)
    return _fixup_tc(out).reshape(R, C, PH, PW)


# speedup vs baseline: 1.3807x; 1.3807x over previous
import functools

import jax
import jax.numpy as jnp
from jax import lax
from jax.experimental import pallas as pl
from jax.experimental.pallas import tpu as pltpu
from jax.experimental.pallas import tpu_sc as plsc

N, C, H, W = 2, 256, 100, 100
R = 512
PH = PW = 7
SR = 2
SCALE = 0.25

NCORES = 2
NSUB = 16
NW = NCORES * NSUB
RPW = R // NW
NSAMP = PH * SR
NBIN = PH * PW
NCHUNK = 13
CV = C // 16


TOT = RPW * NCHUNK


def _roi_align_body(flat, roisp, out, roi_v, bref, ylw, yhw, xli,
                    lyf, hyf, lxf, hxf, wbuf, rows, acc, sems, sem_out):
    cid = lax.axis_index("c")
    sid = lax.axis_index("s")
    wid = sid * NCORES + cid
    roi_base = wid * RPW

    pltpu.sync_copy(roisp.at[wid], roi_v)

    io = lax.iota(jnp.int32, 16)
    iof = io.astype(jnp.float32)
    coeff = iof * 0.5 + 0.25
    lane_ok = io < NSAMP

    def splat(v):
        return jnp.full((16,), v, jnp.int32)

    def axis_quantities(start, binsz, extent):
        ss = start + coeff * binsz
        valid = (ss >= -1.0) & (ss <= float(extent)) & lane_ok
        s0 = jnp.maximum(ss, 0.0)
        lo = jnp.minimum(s0.astype(jnp.int32), extent - 1)
        hi = jnp.minimum(lo + 1, extent - 1)
        frac = jnp.minimum(s0, float(extent - 1)) - lo.astype(jnp.float32)
        lofrac = jnp.where(valid, 1.0 - frac, 0.0)
        hifrac = jnp.where(valid, frac, 0.0)
        return lo, hi, lofrac, hifrac

    def setup_roi(r, _):
        def field(i):
            return plsc.load_gather(roi_v, [splat(i), splat(r)])

        b_v = field(0).astype(jnp.int32) * (H * W)
        sx_s = field(1) * SCALE - 0.5
        sy_s = field(2) * SCALE - 0.5
        ex_s = field(3) * SCALE - 0.5
        ey_s = field(4) * SCALE - 0.5
        bw_s = (ex_s - sx_s) / PW
        bh_s = (ey_s - sy_s) / PH
        yl_v, yh_v, hy_v, ly_v = axis_quantities(sy_s, bh_s, H)
        xl_v, _xh_v, hx_v, lx_v = axis_quantities(sx_s, bw_s, W)
        bref[r] = b_v
        ylw[r] = yl_v * W
        yhw[r] = yh_v * W
        xli[r] = xl_v
        lyf[r] = ly_v
        hyf[r] = hy_v
        lxf[r] = lx_v * 0.25
        hxf[r] = hx_v * 0.25
        return 0

    lax.fori_loop(0, RPW, setup_roi, 0)

    def rr_q(k):
        rr = (k * 5042) >> 16
        return rr, k - rr * NCHUNK

    def issue(k, slot):
        rr, q = rr_q(k)
        rsp = splat(rr)
        b_v = bref[rr]
        s = q * 16 + io
        bin_ = s >> 2
        rem = s & 3
        py = (bin_ * 9363) >> 16
        px = bin_ - py * 7
        ky = py * 2 + (rem >> 1)
        kx = px * 2 + (rem & 1)
        ylw_s = plsc.load_gather(ylw, [rsp, ky])
        yhw_s = plsc.load_gather(yhw, [rsp, ky])
        hy_s = plsc.load_gather(hyf, [rsp, ky])
        ly_s = plsc.load_gather(lyf, [rsp, ky])
        xl_s = plsc.load_gather(xli, [rsp, kx])
        hx_s = plsc.load_gather(hxf, [rsp, kx])
        lx_s = plsc.load_gather(lxf, [rsp, kx])
        wbuf[slot, 0] = hy_s * hx_s
        wbuf[slot, 1] = hy_s * lx_s
        wbuf[slot, 2] = ly_s * hx_s
        wbuf[slot, 3] = ly_s * lx_s
        ilo = b_v + ylw_s + xl_s
        ihi = b_v + yhw_s + xl_s
        sem = sems.at[slot]
        pltpu.make_async_copy(flat.at[ilo], rows.at[slot, pl.ds(0, 16)], sem).start()
        pltpu.make_async_copy(flat.at[ihi], rows.at[slot, pl.ds(16, 16)], sem).start()

    sh16 = jnp.full((16,), 16, jnp.int32)
    himask = jnp.full((16,), -65536, jnp.int32)

    def step(k, _):
        slot = k & 3
        rr, q = rr_q(k)
        aslot = rr & 1

        @pl.when(k < TOT - 3)
        def _():
            issue(k + 3, (k + 3) & 3)

        @pl.when((q == 0) & (rr >= 2))
        def _():
            pltpu.make_async_copy(acc.at[aslot], out.at[roi_base + rr],
                                  sem_out).wait()

        pltpu.make_async_copy(flat.at[pl.ds(0, 32)], rows.at[slot],
                              sems.at[slot]).wait()

        for t in range(4):
            acce = [jnp.zeros((16,), jnp.float32) for _ in range(8)]
            acco = [jnp.zeros((16,), jnp.float32) for _ in range(8)]
            for i in range(4):
                w1 = plsc.load_gather(wbuf, [splat(slot), splat(0), splat(t * 4 + i)])
                w2 = plsc.load_gather(wbuf, [splat(slot), splat(1), splat(t * 4 + i)])
                w3 = plsc.load_gather(wbuf, [splat(slot), splat(2), splat(t * 4 + i)])
                w4 = plsc.load_gather(wbuf, [splat(slot), splat(3), splat(t * 4 + i)])
                rbl = t * 4 + i
                rbh = 16 + t * 4 + i
                for g in range(8):
                    u1 = rows[slot, rbl, pl.ds(g * 16, 16)]
                    u2 = rows[slot, rbl, pl.ds(128 + g * 16, 16)]
                    u3 = rows[slot, rbh, pl.ds(g * 16, 16)]
                    u4 = rows[slot, rbh, pl.ds(128 + g * 16, 16)]
                    acce[g] = (acce[g]
                               + w1 * plsc.bitcast(u1 << sh16, jnp.float32)
                               + w2 * plsc.bitcast(u2 << sh16, jnp.float32)
                               + w3 * plsc.bitcast(u3 << sh16, jnp.float32)
                               + w4 * plsc.bitcast(u4 << sh16, jnp.float32))
                    acco[g] = (acco[g]
                               + w1 * plsc.bitcast(u1 & himask, jnp.float32)
                               + w2 * plsc.bitcast(u2 & himask, jnp.float32)
                               + w3 * plsc.bitcast(u3 & himask, jnp.float32)
                               + w4 * plsc.bitcast(u4 & himask, jnp.float32))
            binrow = q * 4 + t
            for g in range(8):
                acc[aslot, binrow, pl.ds(g * 16, 16)] = acce[g]
                acc[aslot, binrow, pl.ds(128 + g * 16, 16)] = acco[g]

        @pl.when(q == NCHUNK - 1)
        def _():
            pltpu.make_async_copy(acc.at[aslot], out.at[roi_base + rr],
                                  sem_out).start()

        return 0

    issue(0, 0)
    issue(1, 1)
    issue(2, 2)
    lax.fori_loop(0, TOT, step, 0)
    pltpu.make_async_copy(acc.at[0], out.at[roi_base], sem_out).wait()
    pltpu.make_async_copy(acc.at[1], out.at[roi_base], sem_out).wait()


def _prep_body(in_ref, out_ref):
    t = in_ref[0].reshape(C, H * W).T.astype(jnp.bfloat16)
    lo = lax.bitcast_convert_type(t[:, :C // 2], jnp.uint16).astype(jnp.int32)
    hi = lax.bitcast_convert_type(t[:, C // 2:], jnp.uint16).astype(jnp.int32)
    packed = lo | (hi << 16)
    out_ref[:, :C // 2] = packed
    out_ref[:, C // 2:] = jnp.concatenate([packed[1:], packed[:1]], axis=0)


@jax.jit
def _prep_tc(x):
    return pl.pallas_call(
        _prep_body,
        grid=(N,),
        in_specs=[pl.BlockSpec((1, C, H, W), lambda b: (b, 0, 0, 0))],
        out_specs=pl.BlockSpec((H * W, C), lambda b: (b, 0)),
        out_shape=jax.ShapeDtypeStruct((N * H * W, C), jnp.int32),
    )(x)


@jax.jit
def _roi_align_sc(flat, roisp):
    mesh = plsc.VectorSubcoreMesh(core_axis_name="c", subcore_axis_name="s",
                                  num_cores=NCORES, num_subcores=NSUB)
    run = pl.kernel(
        _roi_align_body,
        out_type=jax.ShapeDtypeStruct((R, 52, C), jnp.float32),
        mesh=mesh,
        scratch_types=[
            pltpu.VMEM((5, RPW), jnp.float32),
            pltpu.VMEM((RPW, 16), jnp.int32),
            pltpu.VMEM((RPW, 16), jnp.int32),
            pltpu.VMEM((RPW, 16), jnp.int32),
            pltpu.VMEM((RPW, 16), jnp.int32),
            pltpu.VMEM((RPW, 16), jnp.float32),
            pltpu.VMEM((RPW, 16), jnp.float32),
            pltpu.VMEM((RPW, 16), jnp.float32),
            pltpu.VMEM((RPW, 16), jnp.float32),
            pltpu.VMEM((4, 4, 16), jnp.float32),
            pltpu.VMEM((4, 32, C), jnp.int32),
            pltpu.VMEM((2, 52, C), jnp.float32),
            pltpu.SemaphoreType.DMA((4,)),
            pltpu.SemaphoreType.DMA,
        ],
        compiler_params=pltpu.CompilerParams(needs_layout_passes=False),
    )
    return run(flat, roisp)


def kernel(input, rois):
    flat = _prep_tc(input)
    roisp = jnp.transpose(rois, (1, 0)).reshape(5, NW, RPW).transpose(1, 0, 2)
    out = _roi_align_sc(flat, roisp)
    return out[:, :NBIN].reshape(R, PH, PW, C).transpose(0, 3, 1, 2)
